# baseline (speedup 1.0000x reference)
import functools

import jax
import jax.numpy as jnp
from jax import lax
from jax.experimental import pallas as pl
from jax.experimental.pallas import tpu as pltpu

_N = 10000
_G = 256
_EP = 32768


def _msg_body(ea_ref, xg_ref, w1_ref, b1_ref, wr_ref, br_ref, out_ref, *, K, O):
    h = jnp.maximum(
        jnp.dot(ea_ref[...], w1_ref[...], preferred_element_type=jnp.float32, precision=lax.Precision.HIGHEST)
        + b1_ref[...], 0.0)
    T = jnp.dot(xg_ref[...], wr_ref[...], preferred_element_type=jnp.float32, precision=lax.Precision.HIGHEST)
    acc = jnp.dot(xg_ref[...], br_ref[...], preferred_element_type=jnp.float32, precision=lax.Precision.HIGHEST)
    for k in range(K):
        acc = acc + h[:, k:k + 1] * T[:, k * O:(k + 1) * O]
    out_ref[...] = acc


def _edge_messages(ea, xg, w1, b1, wr, br, *, K, O, Be):
    EP, IN = xg.shape
    ECH = ea.shape[1]
    grid = (EP // Be,)
    return pl.pallas_call(
        functools.partial(_msg_body, K=K, O=O),
        grid=grid,
        in_specs=[
            pl.BlockSpec((Be, ECH), lambda i: (i, 0)),
            pl.BlockSpec((Be, IN), lambda i: (i, 0)),
            pl.BlockSpec((ECH, K), lambda i: (0, 0)),
            pl.BlockSpec((1, K), lambda i: (0, 0)),
            pl.BlockSpec((IN, K * O), lambda i: (0, 0)),
            pl.BlockSpec((IN, O), lambda i: (0, 0)),
        ],
        out_specs=pl.BlockSpec((Be, O), lambda i: (i, 0)),
        out_shape=jax.ShapeDtypeStruct((EP, O), jnp.float32),
    )(ea, xg, w1, b1, wr, br)


def _node_body(agg_ref, cnt_ref, xin_ref, root_ref, bias_ref, g_ref, b_ref,
               out_ref):
    cnt = jnp.clip(cnt_ref[...][:, :1], 1.0, None)
    h = (agg_ref[...] / cnt
         + jnp.dot(xin_ref[...], root_ref[...],
                   preferred_element_type=jnp.float32, precision=lax.Precision.HIGHEST)
         + bias_ref[...])
    h = jnp.maximum(h, 0.0)
    m = jnp.mean(h, axis=0, keepdims=True)
    v = jnp.mean((h - m) ** 2, axis=0, keepdims=True)
    out_ref[...] = (h - m) / jnp.sqrt(v + 1e-5) * g_ref[...] + b_ref[...]


def _node_update(agg, cnt, xin, root, bias, g, b):
    n, O = agg.shape
    return pl.pallas_call(
        _node_body,
        out_shape=jax.ShapeDtypeStruct((n, O), jnp.float32),
    )(agg, cnt, xin, root, bias, g, b)


def _final_body(agg_ref, cnt_ref, hin_ref, root_ref, bias_ref, g_ref, b_ref,
                batch_ref, l1w_ref, l1b_ref, l2w_ref, l2b_ref, out_ref):
    cnt = jnp.clip(cnt_ref[...][:, :1], 1.0, None)
    h = (agg_ref[...] / cnt
         + jnp.dot(hin_ref[...], root_ref[...],
                   preferred_element_type=jnp.float32, precision=lax.Precision.HIGHEST)
         + bias_ref[...])
    h = jnp.maximum(h, 0.0)
    m = jnp.mean(h, axis=0, keepdims=True)
    v = jnp.mean((h - m) ** 2, axis=0, keepdims=True)
    h = (h - m) / jnp.sqrt(v + 1e-5) * g_ref[...] + b_ref[...]
    gids = lax.broadcasted_iota(jnp.int32, (_G, 1), 0)
    oh = (gids == batch_ref[...]).astype(jnp.float32)
    pooled = jnp.dot(oh, h, preferred_element_type=jnp.float32, precision=lax.Precision.HIGHEST)
    gcnt = jnp.clip(jnp.sum(oh, axis=1, keepdims=True), 1.0, None)
    pooled = pooled / gcnt
    z = jnp.maximum(
        jnp.dot(pooled, l1w_ref[...], preferred_element_type=jnp.float32, precision=lax.Precision.HIGHEST)
        + l1b_ref[...], 0.0)
    out_ref[...] = (jnp.dot(z, l2w_ref[...],
                            preferred_element_type=jnp.float32, precision=lax.Precision.HIGHEST) + l2b_ref[...])


def _final_stage(agg, cnt, hin, root, bias, g, b, batch_row, l1w, l1b, l2w, l2b):
    return pl.pallas_call(
        _final_body,
        out_shape=jax.ShapeDtypeStruct((_G, 1), jnp.float32),
    )(agg, cnt, hin, root, bias, g, b, batch_row, l1w, l1b, l2w, l2b)


def _gather_rows(table, idx):
    return jnp.take(table, idx, axis=0)


def _scatter_add(msg, dst, with_counts):
    agg = jax.ops.segment_sum(msg, dst, num_segments=_N + 1)[:_N]
    if with_counts:
        cnt = jax.ops.segment_sum(jnp.ones((dst.shape[0],), jnp.float32), dst,
                                  num_segments=_N + 1)[:_N]
        return agg, cnt.reshape(_N, 1)
    return agg, None


def kernel(x, edge_index, edge_attr, batch,
           nn1_w1, nn1_b1, nn1_w2, nn1_b2, root1, bias1, bn1_g, bn1_b,
           nn2_w1, nn2_b1, nn2_w2, nn2_b2, root2, bias2, bn2_g, bn2_b,
           lin1_w, lin1_b, lin2_w, lin2_b):
    E = edge_index.shape[1]
    src = edge_index[0]
    dst = edge_index[1]
    pad = _EP - E
    src_p = jnp.concatenate([src, jnp.zeros((pad,), jnp.int32)])
    dst_p = jnp.concatenate([dst, jnp.full((pad,), _N, jnp.int32)])
    ea_p = jnp.concatenate([edge_attr,
                            jnp.zeros((pad, edge_attr.shape[1]), jnp.float32)])

    K = 32
    w1r2 = nn1_w2.reshape(K, 32, 64).transpose(1, 0, 2).reshape(32, K * 64)
    b1r = nn1_b2.reshape(32, 64)
    w2r2 = nn2_w2.reshape(K, 64, 128).transpose(1, 0, 2).reshape(64, K * 128)
    b2r = nn2_b2.reshape(64, 128)

    xg1 = _gather_rows(x, src_p)
    msg1 = _edge_messages(ea_p, xg1, nn1_w1, nn1_b1.reshape(1, K),
                          w1r2, b1r, K=K, O=64, Be=512)
    agg1, cnt = _scatter_add(msg1, dst_p, with_counts=True)
    h1 = _node_update(agg1, cnt, x, root1, bias1.reshape(1, 64),
                      bn1_g.reshape(1, 64), bn1_b.reshape(1, 64))

    xg2 = _gather_rows(h1, src_p)
    msg2 = _edge_messages(ea_p, xg2, nn2_w1, nn2_b1.reshape(1, K),
                          w2r2, b2r, K=K, O=128, Be=256)
    agg2, _ = _scatter_add(msg2, dst_p, with_counts=False)

    out = _final_stage(agg2, cnt, h1, root2, bias2.reshape(1, 128),
                       bn2_g.reshape(1, 128), bn2_b.reshape(1, 128),
                       batch.reshape(1, _N), lin1_w, lin1_b.reshape(1, 64),
                       lin2_w, lin2_b.reshape(1, 1))
    return out.reshape(_G)


# speedup vs baseline: 1.5852x; 1.5852x over previous
import functools

import jax
import jax.numpy as jnp
from jax import lax
from jax.experimental import pallas as pl
from jax.experimental.pallas import tpu as pltpu
from jax.experimental.pallas import tpu_sc as plsc

_N = 10000
_G = 256
_NW = 32
_KJ = 8
_B = 128
_EP = _NW * _KJ * _B
_NP = _N + 16
_STR = _NP // 16
_CW = 16

_HIGH = lax.Precision.HIGHEST


def _sc_gather(table, idx3d, D):
    mesh = plsc.VectorSubcoreMesh(core_axis_name="c", subcore_axis_name="s")

    @functools.partial(
        pl.kernel,
        out_type=jax.ShapeDtypeStruct((_NW, _KJ, _B, D), jnp.float32),
        mesh=mesh,
        compiler_params=pltpu.CompilerParams(use_tc_tiling_on_sc=False),
        scratch_types=[
            pltpu.VMEM((_KJ, _B), jnp.int32),
            pltpu.VMEM((_KJ, _B, D), jnp.float32),
            pltpu.SemaphoreType.DMA,
        ],
    )
    def gather_kernel(table_hbm, idx_hbm, out_hbm, idx_v, rows_v, sem):
        wid = lax.axis_index("s") * 2 + lax.axis_index("c")
        pltpu.sync_copy(idx_hbm.at[wid], idx_v)
        copies = [
            pltpu.async_copy(table_hbm.at[idx_v.at[j]], rows_v.at[j], sem)
            for j in range(_KJ)
        ]
        for cp in copies:
            cp.wait()
        pltpu.sync_copy(rows_v, out_hbm.at[wid])

    return gather_kernel(table, idx3d)


def _zero_rows(buf, nrows, width):
    z = jnp.zeros((16,), jnp.float32)

    def row(r, carry):
        for cc in range(width // 16):
            buf[r, pl.ds(cc * 16, 16)] = z
        return carry

    lax.fori_loop(0, nrows, row, 0)


def _sc_scatter(msg4d, idx3d, O, with_counts):
    mesh = plsc.VectorSubcoreMesh(core_axis_name="c", subcore_axis_name="s")

    out_type = [jax.ShapeDtypeStruct((2, _NP, O), jnp.float32)]
    scratch = [
        pltpu.VMEM((_KJ, _B), jnp.int32),
        pltpu.VMEM((_B, O), jnp.float32),
        pltpu.VMEM_SHARED((_NP, O), jnp.float32),
    ]
    if with_counts:
        out_type.append(jax.ShapeDtypeStruct((2, _NP, _CW), jnp.float32))
        scratch += [
            pltpu.VMEM((_B, _CW), jnp.float32),
            pltpu.VMEM_SHARED((_NP, _CW), jnp.float32),
        ]
    chunks = []
    off = 0
    while off < _STR:
        n = min(_B, _STR - off)
        chunks.append((off, n))
        off += n

    @functools.partial(pl.kernel, out_type=tuple(out_type), mesh=mesh,
                       compiler_params=pltpu.CompilerParams(
                           use_tc_tiling_on_sc=False),
                       scratch_types=tuple(scratch))
    def scatter_kernel(msg_hbm, idx_hbm, *refs):
        if with_counts:
            (agg_out, cnt_out, idx_v, msg_v, agg_sh, ones_v, cnt_sh) = refs
        else:
            agg_out, idx_v, msg_v, agg_sh = refs
        c = lax.axis_index("c")
        s = lax.axis_index("s")
        wid = s * 2 + c

        _zero_rows(msg_v, _B, O)
        for off, n in chunks:
            pltpu.sync_copy(msg_v.at[pl.ds(0, n)],
                            agg_sh.at[pl.ds(s * _STR + off, n)])
        if with_counts:
            _zero_rows(ones_v, _B, _CW)
            for off, n in chunks:
                pltpu.sync_copy(ones_v.at[pl.ds(0, n)],
                                cnt_sh.at[pl.ds(s * _STR + off, n)])
            one = jnp.ones((16,), jnp.float32)

            def orow(r, carry):
                ones_v[r, pl.ds(0, 16)] = one
                return carry

            lax.fori_loop(0, _B, orow, 0)
        plsc.subcore_barrier()

        pltpu.sync_copy(idx_hbm.at[wid], idx_v)
        for j in range(_KJ):
            pltpu.sync_copy(msg_hbm.at[wid, j], msg_v)
            pltpu.sync_copy(msg_v, agg_sh.at[idx_v.at[j]], add=True)
            if with_counts:
                pltpu.sync_copy(ones_v, cnt_sh.at[idx_v.at[j]], add=True)
        plsc.subcore_barrier()

        pltpu.sync_copy(agg_sh.at[pl.ds(s * _STR, _STR)],
                        agg_out.at[c, pl.ds(s * _STR, _STR)])
        if with_counts:
            pltpu.sync_copy(cnt_sh.at[pl.ds(s * _STR, _STR)],
                            cnt_out.at[c, pl.ds(s * _STR, _STR)])

    res = scatter_kernel(msg4d, idx3d)
    if with_counts:
        return res[0], res[1]
    return res[0], None


def _msg_body(ea_ref, xg_ref, w1_ref, b1_ref, whi_ref, wlo_ref, out_ref, *,
              K, O):
    h = jnp.maximum(
        jnp.dot(ea_ref[...], w1_ref[...], preferred_element_type=jnp.float32,
                precision=_HIGH) + b1_ref[...], 0.0)
    xg = xg_ref[...]
    hi_parts, lo_parts = [], []
    for part in [h[:, k:k + 1] * xg for k in range(K)] + [xg]:
        p_hi = part.astype(jnp.bfloat16)
        hi_parts.append(p_hi)
        lo_parts.append((part - p_hi.astype(jnp.float32)).astype(jnp.bfloat16))
    P_hi = jnp.concatenate(hi_parts, axis=1)
    P_lo = jnp.concatenate(lo_parts, axis=1)
    acc = jnp.dot(P_hi, whi_ref[...], preferred_element_type=jnp.float32)
    acc += jnp.dot(P_hi, wlo_ref[...], preferred_element_type=jnp.float32)
    acc += jnp.dot(P_lo, whi_ref[...], preferred_element_type=jnp.float32)
    out_ref[...] = acc


def _edge_messages(ea, xg, w1, b1, wcat_hi, wcat_lo, *, K, O, Be):
    EP, IN = xg.shape
    ECH = ea.shape[1]
    return pl.pallas_call(
        functools.partial(_msg_body, K=K, O=O),
        grid=(EP // Be,),
        in_specs=[
            pl.BlockSpec((Be, ECH), lambda i: (i, 0)),
            pl.BlockSpec((Be, IN), lambda i: (i, 0)),
            pl.BlockSpec((ECH, K), lambda i: (0, 0)),
            pl.BlockSpec((1, K), lambda i: (0, 0)),
            pl.BlockSpec(((K + 1) * IN, O), lambda i: (0, 0)),
            pl.BlockSpec(((K + 1) * IN, O), lambda i: (0, 0)),
        ],
        out_specs=pl.BlockSpec((Be, O), lambda i: (i, 0)),
        out_shape=jax.ShapeDtypeStruct((EP, O), jnp.float32),
    )(ea, xg, w1, b1, wcat_hi, wcat_lo)


def _bn_update(agg2_ref, cnt2_ref, xin, root_ref, bias_ref, g_ref, b_ref):
    agg = agg2_ref[0, : _N, :] + agg2_ref[1, : _N, :]
    cnt = jnp.clip(cnt2_ref[0, : _N, :1] + cnt2_ref[1, : _N, :1], 1.0, None)
    h = (agg / cnt
         + jnp.dot(xin, root_ref[...], preferred_element_type=jnp.float32,
                   precision=_HIGH)
         + bias_ref[...])
    h = jnp.maximum(h, 0.0)
    m = jnp.mean(h, axis=0, keepdims=True)
    v = jnp.mean((h - m) ** 2, axis=0, keepdims=True)
    return (h - m) / jnp.sqrt(v + 1e-5) * g_ref[...] + b_ref[...]


def _node_body(agg2_ref, cnt2_ref, xin_ref, root_ref, bias_ref, g_ref, b_ref,
               out_ref):
    out_ref[...] = _bn_update(agg2_ref, cnt2_ref, xin_ref[...], root_ref,
                              bias_ref, g_ref, b_ref)


def _node_update(agg2, cnt2, xin, root, bias, g, b):
    n, O = xin.shape[0], agg2.shape[2]
    return pl.pallas_call(
        _node_body,
        out_shape=jax.ShapeDtypeStruct((n, O), jnp.float32),
    )(agg2, cnt2, xin, root, bias, g, b)


def _final_body(agg2_ref, cnt2_ref, hin_ref, root_ref, bias_ref, g_ref, b_ref,
                batch_ref, l1w_ref, l1b_ref, l2w_ref, l2b_ref, out_ref):
    h = _bn_update(agg2_ref, cnt2_ref, hin_ref[...], root_ref, bias_ref,
                   g_ref, b_ref)
    gids = lax.broadcasted_iota(jnp.int32, (_G, 1), 0)
    oh = (gids == batch_ref[...]).astype(jnp.float32)
    pooled = jnp.dot(oh, h, preferred_element_type=jnp.float32,
                     precision=_HIGH)
    gcnt = jnp.clip(jnp.sum(oh, axis=1, keepdims=True), 1.0, None)
    pooled = pooled / gcnt
    z = jnp.maximum(
        jnp.dot(pooled, l1w_ref[...], preferred_element_type=jnp.float32,
                precision=_HIGH) + l1b_ref[...], 0.0)
    out_ref[...] = (jnp.dot(z, l2w_ref[...], preferred_element_type=jnp.float32,
                            precision=_HIGH) + l2b_ref[...])


def _final_stage(agg2, cnt2, hin, root, bias, g, b, batch_row, l1w, l1b, l2w,
                 l2b):
    return pl.pallas_call(
        _final_body,
        out_shape=jax.ShapeDtypeStruct((_G, 1), jnp.float32),
    )(agg2, cnt2, hin, root, bias, g, b, batch_row, l1w, l1b, l2w, l2b)


def kernel(x, edge_index, edge_attr, batch,
           nn1_w1, nn1_b1, nn1_w2, nn1_b2, root1, bias1, bn1_g, bn1_b,
           nn2_w1, nn2_b1, nn2_w2, nn2_b2, root2, bias2, bn2_g, bn2_b,
           lin1_w, lin1_b, lin2_w, lin2_b):
    E = edge_index.shape[1]
    pad = _EP - E
    src3 = jnp.concatenate([edge_index[0], jnp.zeros((pad,), jnp.int32)]
                           ).reshape(_NW, _KJ, _B)
    dst3 = jnp.concatenate([edge_index[1], jnp.full((pad,), _N, jnp.int32)]
                           ).reshape(_NW, _KJ, _B)
    ea_p = jnp.concatenate(
        [edge_attr, jnp.zeros((pad, edge_attr.shape[1]), jnp.float32)])

    K = 32
    wcat1 = jnp.concatenate(
        [nn1_w2.reshape(K, 32, 64),
         nn1_b2.reshape(1, 32, 64)]).reshape((K + 1) * 32, 64)
    wcat2 = jnp.concatenate(
        [nn2_w2.reshape(K, 64, 128),
         nn2_b2.reshape(1, 64, 128)]).reshape((K + 1) * 64, 128)
    wcat1_hi = wcat1.astype(jnp.bfloat16)
    wcat1_lo = (wcat1 - wcat1_hi.astype(jnp.float32)).astype(jnp.bfloat16)
    wcat2_hi = wcat2.astype(jnp.bfloat16)
    wcat2_lo = (wcat2 - wcat2_hi.astype(jnp.float32)).astype(jnp.bfloat16)

    xg1 = _sc_gather(x, src3, 32).reshape(_EP, 32)
    msg1 = _edge_messages(ea_p, xg1, nn1_w1, nn1_b1.reshape(1, K),
                          wcat1_hi, wcat1_lo, K=K, O=64, Be=512)
    agg1, cnt = _sc_scatter(msg1.reshape(_NW, _KJ, _B, 64), dst3, 64,
                            with_counts=True)
    h1 = _node_update(agg1, cnt, x, root1, bias1.reshape(1, 64),
                      bn1_g.reshape(1, 64), bn1_b.reshape(1, 64))

    xg2 = _sc_gather(h1, src3, 64).reshape(_EP, 64)
    msg2 = _edge_messages(ea_p, xg2, nn2_w1, nn2_b1.reshape(1, K),
                          wcat2_hi, wcat2_lo, K=K, O=128, Be=512)
    agg2, _ = _sc_scatter(msg2.reshape(_NW, _KJ, _B, 128), dst3, 128,
                          with_counts=False)

    out = _final_stage(agg2, cnt, h1, root2, bias2.reshape(1, 128),
                       bn2_g.reshape(1, 128), bn2_b.reshape(1, 128),
                       batch.reshape(1, _N), lin1_w, lin1_b.reshape(1, 64),
                       lin2_w, lin2_b.reshape(1, 1))
    return out.reshape(_G)
